# baseline (speedup 1.0000x reference)
import functools

import jax
import jax.numpy as jnp
from jax import lax
from jax.experimental import pallas as pl
from jax.experimental.pallas import tpu as pltpu
from jax.experimental.pallas import tpu_sc as plsc

HID = 256
HEADS = 4
DH = HID // HEADS
N_TERM = 4096
BLK = 256
N_BLKS = N_TERM // BLK
CNTW = 16


def _dense_a_body(sums_ref, cnts_ref, x_ref, wl_ref, wr_ref, bsum_ref,
                  lng_ref, lnb_ref, inw_ref, inb_ref,
                  h_ref, q_ref, k_ref, v_ref):
    x = x_ref[...]
    acc = x + bsum_ref[...]
    acc = acc + lax.dot_general(x, wr_ref[...], (((1,), (1,)), ((), ())),
                                preferred_element_type=jnp.float32)
    for r in range(4):
        c = jnp.transpose(cnts_ref[r, 0, :, 0:BLK])
        inv = 1.0 / jnp.maximum(c, 1.0)
        mean = sums_ref[r] * inv
        acc = acc + lax.dot_general(mean, wl_ref[r], (((1,), (1,)), ((), ())),
                                    preferred_element_type=jnp.float32)
    mu = jnp.mean(acc, axis=1, keepdims=True)
    var = jnp.mean(acc * acc, axis=1, keepdims=True) - mu * mu
    h = (acc - mu) * lax.rsqrt(var + 1e-5) * lng_ref[...] + lnb_ref[...]
    h_ref[...] = h
    qkv = lax.dot_general(h, inw_ref[...], (((1,), (1,)), ((), ())),
                          preferred_element_type=jnp.float32) + inb_ref[...]
    q_ref[...] = qkv[:, 0:HID]
    k_ref[...] = qkv[:, HID:2 * HID]
    v_ref[...] = qkv[:, 2 * HID:3 * HID]


def _dense_a(sums, cnts, x_term, Wl4, Wr_sum, b_sum, ln_g, ln_b, in_w, in_b):
    spec_row = pl.BlockSpec((BLK, HID), lambda i: (i, 0))
    return pl.pallas_call(
        _dense_a_body,
        grid=(N_BLKS,),
        in_specs=[
            pl.BlockSpec((4, BLK, HID), lambda i: (0, i, 0)),
            pl.BlockSpec((4, 1, 1, ACC_R), lambda i: (0, i, 0, 0)),
            spec_row,
            pl.BlockSpec((4, HID, HID), lambda i: (0, 0, 0)),
            pl.BlockSpec((HID, HID), lambda i: (0, 0)),
            pl.BlockSpec((1, HID), lambda i: (0, 0)),
            pl.BlockSpec((1, HID), lambda i: (0, 0)),
            pl.BlockSpec((1, HID), lambda i: (0, 0)),
            pl.BlockSpec((3 * HID, HID), lambda i: (0, 0)),
            pl.BlockSpec((1, 3 * HID), lambda i: (0, 0)),
        ],
        out_specs=[spec_row, spec_row, spec_row, spec_row],
        out_shape=[jax.ShapeDtypeStruct((N_TERM, HID), jnp.float32)] * 4,
    )(sums, cnts, x_term, Wl4, Wr_sum, b_sum, ln_g, ln_b, in_w, in_b)


def _attn_body(q_ref, k_ref, v_ref, h_ref, ow_ref, ob_ref, pw_ref, pb_ref,
               out_ref):
    q = q_ref[...]
    scale = 1.0 / (DH ** 0.5)
    outs = []
    for hh in range(HEADS):
        qh = q[:, hh * DH:(hh + 1) * DH] * scale
        kh = k_ref[:, hh * DH:(hh + 1) * DH]
        vh = v_ref[:, hh * DH:(hh + 1) * DH]
        s = lax.dot_general(qh, kh, (((1,), (1,)), ((), ())),
                            preferred_element_type=jnp.float32)
        m = jnp.max(s, axis=1, keepdims=True)
        p = jnp.exp(s - m)
        denom = jnp.sum(p, axis=1, keepdims=True)
        attn = p / denom
        outs.append(lax.dot_general(attn, vh, (((1,), (0,)), ((), ())),
                                    preferred_element_type=jnp.float32))
    o = jnp.concatenate(outs, axis=1)
    a = lax.dot_general(o, ow_ref[...], (((1,), (1,)), ((), ())),
                        preferred_element_type=jnp.float32) + ob_ref[...]
    f = lax.dot_general(a, pw_ref[...], (((1,), (1,)), ((), ())),
                        preferred_element_type=jnp.float32) + pb_ref[...]
    out_ref[...] = jnp.maximum(f, 0.0) + h_ref[...]


def _attn(q, k, v, h, out_w, out_b, post_w, post_b):
    spec_row = pl.BlockSpec((BLK, HID), lambda i: (i, 0))
    spec_full = pl.BlockSpec((N_TERM, HID), lambda i: (0, 0))
    return pl.pallas_call(
        _attn_body,
        grid=(N_BLKS,),
        in_specs=[
            spec_row, spec_full, spec_full, spec_row,
            pl.BlockSpec((HID, HID), lambda i: (0, 0)),
            pl.BlockSpec((1, HID), lambda i: (0, 0)),
            pl.BlockSpec((HID, HID), lambda i: (0, 0)),
            pl.BlockSpec((1, HID), lambda i: (0, 0)),
        ],
        out_specs=spec_row,
        out_shape=jax.ShapeDtypeStruct((N_TERM, HID), jnp.float32),
    )(q, k, v, h, out_w, out_b, post_w, post_b)


N_SUB = 16
TROWS = N_TERM // N_SUB
ACC_R = TROWS + 8
SCH = 4096
KSUB = 128


def _sc_body(x_term, x_symbol, x_var, has_src, has_dst, sym_src, sym_dst,
             var_src, var_dst, zacc_h, zcnt_h,
             sums_o, cnts_o,
             sbuf, dbuf, csrc, cdst, rows, acc, cacc, sem):
    sid = lax.axis_index("s")
    cid = lax.axis_index("c")
    lo = sid * TROWS
    lane = lax.iota(jnp.int32, 16)

    def process(r, table, srch, dsth, n_edges):
        pltpu.sync_copy(zacc_h, acc)
        pltpu.sync_copy(zcnt_h, cacc)
        def zbody(i, carry):
            csrc[pl.ds(i * 16, 16)] = jnp.zeros((16,), jnp.int32)
            return carry
        lax.fori_loop(0, (SCH + KSUB) // 16, zbody, 0)

        def schunk(s, carry):
            pltpu.sync_copy(srch.at[pl.ds(s * SCH, SCH)], sbuf)
            pltpu.sync_copy(dsth.at[pl.ds(s * SCH, SCH)], dbuf)

            def fbody(g, cur):
                dvec = dbuf[pl.ds(g * 16, 16)]
                svec = sbuf[pl.ds(g * 16, 16)]
                loc = dvec - lo
                m = (loc >= 0) & (loc < TROWS)
                locc = jnp.where(m, loc, TROWS)
                cnt, last = plsc.scan_count(locc)
                plsc.addupdate_scatter(cacc, [locc],
                                       cnt.astype(jnp.float32), mask=last)
                mi = m.astype(jnp.int32)
                pos = cur + plsc.cumsum(mi) - 1
                plsc.store_scatter(csrc, [pos], svec, mask=m)
                plsc.store_scatter(cdst, [pos], loc, mask=m)
                return cur + jnp.sum(mi)

            cur = lax.fori_loop(0, SCH // 16, fbody, 0)

            def kbody(k, carry):
                pltpu.async_copy(table.at[csrc.at[pl.ds(k * KSUB, KSUB)]],
                                 rows, sem).wait()
                base = k * KSUB

                def gbody(g, gcarry):
                    dvec = cdst[pl.ds(base + g * 16, 16)]
                    valid = (base + g * 16 + lane) < cur
                    loc = jnp.where(valid, dvec, TROWS)
                    for j in range(16):
                        d = jnp.sum(jnp.where(lane == j, loc, 0))
                        e = g * 16 + j
                        for q in range(HID // 16):
                            acc[d, pl.ds(q * 16, 16)] = (
                                acc[d, pl.ds(q * 16, 16)]
                                + rows[e, pl.ds(q * 16, 16)])
                    return gcarry

                lax.fori_loop(0, KSUB // 16, gbody, 0)
                return carry

            lax.fori_loop(0, (cur + KSUB - 1) // KSUB, kbody, 0)
            return carry

        lax.fori_loop(0, n_edges // SCH, schunk, 0)

        pltpu.sync_copy(acc.at[pl.ds(0, TROWS)],
                        sums_o.at[r, pl.ds(lo, TROWS)])
        pltpu.sync_copy(cacc, cnts_o.at[r, sid, 0])

    @pl.when(cid == 0)
    def _():
        process(0, x_term, has_dst, has_src, has_src.shape[0])
        process(2, x_symbol, sym_dst, sym_src, sym_src.shape[0])

    @pl.when(cid == 1)
    def _():
        process(1, x_term, has_src, has_dst, has_src.shape[0])
        process(3, x_var, var_src, var_dst, var_src.shape[0])


_sc_call = functools.partial(
    pl.kernel,
    out_type=[
        jax.ShapeDtypeStruct((4, N_TERM, HID), jnp.float32),
        jax.ShapeDtypeStruct((4, N_SUB, 1, ACC_R), jnp.float32),
    ],
    mesh=plsc.VectorSubcoreMesh(core_axis_name="c", subcore_axis_name="s"),
    compiler_params=pltpu.CompilerParams(needs_layout_passes=False),
    scratch_types=[
        pltpu.VMEM((SCH,), jnp.int32),
        pltpu.VMEM((SCH,), jnp.int32),
        pltpu.VMEM((SCH + KSUB,), jnp.int32),
        pltpu.VMEM((SCH + KSUB,), jnp.int32),
        pltpu.VMEM((KSUB, HID), jnp.float32),
        pltpu.VMEM((ACC_R, HID), jnp.float32),
        pltpu.VMEM((ACC_R,), jnp.float32),
        pltpu.SemaphoreType.DMA,
    ],
)(_sc_body)


def _segment_sums(x_term, x_symbol, x_var, has_arg_src, has_arg_dst,
                  symbol_of_src, symbol_of_dst, var_occ_src, var_occ_dst):
    zacc_h = jnp.zeros((ACC_R, HID), jnp.float32)
    zcnt_h = jnp.zeros((ACC_R,), jnp.float32)
    return _sc_call(x_term, x_symbol, x_var, has_arg_src, has_arg_dst,
                    symbol_of_src, symbol_of_dst, var_occ_src, var_occ_dst,
                    zacc_h, zcnt_h)


def kernel(x_term, x_symbol, x_var, has_arg_src, has_arg_dst, symbol_of_src,
           symbol_of_dst, var_occ_src, var_occ_dst, Wl, bl, Wr, ln_g, ln_b,
           attn_in_w, attn_in_b, attn_out_w, attn_out_b, post_w, post_b):
    sums, cnts = _segment_sums(x_term, x_symbol, x_var, has_arg_src,
                               has_arg_dst, symbol_of_src, symbol_of_dst,
                               var_occ_src, var_occ_dst)
    Wr_sum = Wr[0] + Wr[1] + Wr[2] + Wr[3]
    b_sum = (bl[0] + bl[1] + bl[2] + bl[3]).reshape(1, HID)
    h, q, k, v = _dense_a(sums, cnts, x_term, Wl[:4], Wr_sum, b_sum,
                          ln_g.reshape(1, HID), ln_b.reshape(1, HID),
                          attn_in_w, attn_in_b.reshape(1, 3 * HID))
    return _attn(q, k, v, h, attn_out_w, attn_out_b.reshape(1, HID),
                 post_w, post_b.reshape(1, HID))


# speedup vs baseline: 1.0044x; 1.0044x over previous
import functools

import jax
import jax.numpy as jnp
from jax import lax
from jax.experimental import pallas as pl
from jax.experimental.pallas import tpu as pltpu
from jax.experimental.pallas import tpu_sc as plsc

HID = 256
HEADS = 4
DH = HID // HEADS
N_TERM = 4096
BLK = 256
N_BLKS = N_TERM // BLK
CNTW = 16


def _dense_a_body(sums_ref, cnts_ref, x_ref, wl_ref, wr_ref, bsum_ref,
                  lng_ref, lnb_ref, inw_ref, inb_ref,
                  h_ref, q_ref, k_ref, v_ref):
    x = x_ref[...]
    acc = x + bsum_ref[...]
    acc = acc + lax.dot_general(x, wr_ref[...], (((1,), (1,)), ((), ())),
                                preferred_element_type=jnp.float32)
    for r in range(4):
        c = jnp.transpose(cnts_ref[r, 0, :, 0:BLK])
        inv = 1.0 / jnp.maximum(c, 1.0)
        mean = sums_ref[r] * inv
        acc = acc + lax.dot_general(mean, wl_ref[r], (((1,), (1,)), ((), ())),
                                    preferred_element_type=jnp.float32)
    mu = jnp.mean(acc, axis=1, keepdims=True)
    var = jnp.mean(acc * acc, axis=1, keepdims=True) - mu * mu
    h = (acc - mu) * lax.rsqrt(var + 1e-5) * lng_ref[...] + lnb_ref[...]
    h_ref[...] = h
    qkv = lax.dot_general(h, inw_ref[...], (((1,), (1,)), ((), ())),
                          preferred_element_type=jnp.float32) + inb_ref[...]
    q_ref[...] = qkv[:, 0:HID]
    k_ref[...] = qkv[:, HID:2 * HID]
    v_ref[...] = qkv[:, 2 * HID:3 * HID]


def _dense_a(sums, cnts, x_term, Wl4, Wr_sum, b_sum, ln_g, ln_b, in_w, in_b):
    spec_row = pl.BlockSpec((BLK, HID), lambda i: (i, 0))
    return pl.pallas_call(
        _dense_a_body,
        grid=(N_BLKS,),
        in_specs=[
            pl.BlockSpec((4, BLK, HID), lambda i: (0, i, 0)),
            pl.BlockSpec((4, 1, 1, ACC_R), lambda i: (0, i, 0, 0)),
            spec_row,
            pl.BlockSpec((4, HID, HID), lambda i: (0, 0, 0)),
            pl.BlockSpec((HID, HID), lambda i: (0, 0)),
            pl.BlockSpec((1, HID), lambda i: (0, 0)),
            pl.BlockSpec((1, HID), lambda i: (0, 0)),
            pl.BlockSpec((1, HID), lambda i: (0, 0)),
            pl.BlockSpec((3 * HID, HID), lambda i: (0, 0)),
            pl.BlockSpec((1, 3 * HID), lambda i: (0, 0)),
        ],
        out_specs=[spec_row, spec_row, spec_row, spec_row],
        out_shape=[jax.ShapeDtypeStruct((N_TERM, HID), jnp.float32)] * 4,
    )(sums, cnts, x_term, Wl4, Wr_sum, b_sum, ln_g, ln_b, in_w, in_b)


def _attn_body(q_ref, k_ref, v_ref, h_ref, ow_ref, ob_ref, pw_ref, pb_ref,
               out_ref):
    q = q_ref[...]
    scale = 1.0 / (DH ** 0.5)
    outs = []
    for hh in range(HEADS):
        qh = q[:, hh * DH:(hh + 1) * DH] * scale
        kh = k_ref[:, hh * DH:(hh + 1) * DH]
        vh = v_ref[:, hh * DH:(hh + 1) * DH]
        s = lax.dot_general(qh, kh, (((1,), (1,)), ((), ())),
                            preferred_element_type=jnp.float32)
        m = jnp.max(s, axis=1, keepdims=True)
        p = jnp.exp(s - m)
        denom = jnp.sum(p, axis=1, keepdims=True)
        attn = p / denom
        outs.append(lax.dot_general(attn, vh, (((1,), (0,)), ((), ())),
                                    preferred_element_type=jnp.float32))
    o = jnp.concatenate(outs, axis=1)
    a = lax.dot_general(o, ow_ref[...], (((1,), (1,)), ((), ())),
                        preferred_element_type=jnp.float32) + ob_ref[...]
    f = lax.dot_general(a, pw_ref[...], (((1,), (1,)), ((), ())),
                        preferred_element_type=jnp.float32) + pb_ref[...]
    out_ref[...] = jnp.maximum(f, 0.0) + h_ref[...]


def _attn(q, k, v, h, out_w, out_b, post_w, post_b):
    spec_row = pl.BlockSpec((BLK, HID), lambda i: (i, 0))
    spec_full = pl.BlockSpec((N_TERM, HID), lambda i: (0, 0))
    return pl.pallas_call(
        _attn_body,
        grid=(N_BLKS,),
        in_specs=[
            spec_row, spec_full, spec_full, spec_row,
            pl.BlockSpec((HID, HID), lambda i: (0, 0)),
            pl.BlockSpec((1, HID), lambda i: (0, 0)),
            pl.BlockSpec((HID, HID), lambda i: (0, 0)),
            pl.BlockSpec((1, HID), lambda i: (0, 0)),
        ],
        out_specs=spec_row,
        out_shape=jax.ShapeDtypeStruct((N_TERM, HID), jnp.float32),
    )(q, k, v, h, out_w, out_b, post_w, post_b)


N_SUB = 16
TROWS = N_TERM // N_SUB
ACC_R = TROWS + 8
SCH = 4096
KSUB = 128


def _sc_body(x_term, x_symbol, x_var, has_src, has_dst, sym_src, sym_dst,
             var_src, var_dst, zacc_h, zcnt_h,
             sums_o, cnts_o,
             sbuf, dbuf, csrc, cdst, rows, acc, cacc, sem):
    sid = lax.axis_index("s")
    cid = lax.axis_index("c")
    lo = sid * TROWS
    lane = lax.iota(jnp.int32, 16)

    def process(r, table, srch, dsth, n_edges):
        pltpu.sync_copy(zacc_h, acc)
        pltpu.sync_copy(zcnt_h, cacc)
        def zbody(i, carry):
            csrc[pl.ds(i * 16, 16)] = jnp.zeros((16,), jnp.int32)
            return carry
        lax.fori_loop(0, (SCH + KSUB) // 16, zbody, 0)

        def schunk(s, carry):
            pltpu.sync_copy(srch.at[pl.ds(s * SCH, SCH)], sbuf)
            pltpu.sync_copy(dsth.at[pl.ds(s * SCH, SCH)], dbuf)

            def fbody(g, cur):
                dvec = dbuf[pl.ds(g * 16, 16)]
                svec = sbuf[pl.ds(g * 16, 16)]
                loc = dvec - lo
                m = (loc >= 0) & (loc < TROWS)
                locc = jnp.where(m, loc, TROWS)
                cnt, last = plsc.scan_count(locc)
                plsc.addupdate_scatter(cacc, [locc],
                                       cnt.astype(jnp.float32), mask=last)
                mi = m.astype(jnp.int32)
                pos = cur + plsc.cumsum(mi) - 1
                plsc.store_scatter(csrc, [pos], svec, mask=m)
                plsc.store_scatter(cdst, [pos], loc, mask=m)
                return cur + jnp.sum(mi)

            cur = lax.fori_loop(0, SCH // 16, fbody, 0)

            end = ((cur + KSUB - 1) // KSUB) * KSUB
            trash = jnp.full((16,), TROWS, jnp.int32)
            for p in range(KSUB // 16):
                pos = cur + p * 16 + lane
                plsc.store_scatter(cdst, [pos], trash, mask=pos < end)

            def kbody(k, carry):
                pltpu.async_copy(table.at[csrc.at[pl.ds(k * KSUB, KSUB)]],
                                 rows, sem).wait()
                base = k * KSUB

                def gbody(g, gcarry):
                    e0 = base + g * 16
                    dvec = cdst[pl.ds(e0, 16)]
                    for j in range(16):
                        d = dvec[j]
                        e = g * 16 + j
                        for q in range(HID // 16):
                            acc[d, pl.ds(q * 16, 16)] = (
                                acc[d, pl.ds(q * 16, 16)]
                                + rows[e, pl.ds(q * 16, 16)])
                    return gcarry

                lax.fori_loop(0, KSUB // 16, gbody, 0)
                return carry

            lax.fori_loop(0, (cur + KSUB - 1) // KSUB, kbody, 0)
            return carry

        lax.fori_loop(0, n_edges // SCH, schunk, 0)

        pltpu.sync_copy(acc.at[pl.ds(0, TROWS)],
                        sums_o.at[r, pl.ds(lo, TROWS)])
        pltpu.sync_copy(cacc, cnts_o.at[r, sid, 0])

    @pl.when(cid == 0)
    def _():
        process(0, x_term, has_dst, has_src, has_src.shape[0])
        process(2, x_symbol, sym_dst, sym_src, sym_src.shape[0])

    @pl.when(cid == 1)
    def _():
        process(1, x_term, has_src, has_dst, has_src.shape[0])
        process(3, x_var, var_src, var_dst, var_src.shape[0])


_sc_call = functools.partial(
    pl.kernel,
    out_type=[
        jax.ShapeDtypeStruct((4, N_TERM, HID), jnp.float32),
        jax.ShapeDtypeStruct((4, N_SUB, 1, ACC_R), jnp.float32),
    ],
    mesh=plsc.VectorSubcoreMesh(core_axis_name="c", subcore_axis_name="s"),
    compiler_params=pltpu.CompilerParams(needs_layout_passes=False),
    scratch_types=[
        pltpu.VMEM((SCH,), jnp.int32),
        pltpu.VMEM((SCH,), jnp.int32),
        pltpu.VMEM((SCH + KSUB,), jnp.int32),
        pltpu.VMEM((SCH + KSUB,), jnp.int32),
        pltpu.VMEM((KSUB, HID), jnp.float32),
        pltpu.VMEM((ACC_R, HID), jnp.float32),
        pltpu.VMEM((ACC_R,), jnp.float32),
        pltpu.SemaphoreType.DMA,
    ],
)(_sc_body)


def _segment_sums(x_term, x_symbol, x_var, has_arg_src, has_arg_dst,
                  symbol_of_src, symbol_of_dst, var_occ_src, var_occ_dst):
    zacc_h = jnp.zeros((ACC_R, HID), jnp.float32)
    zcnt_h = jnp.zeros((ACC_R,), jnp.float32)
    return _sc_call(x_term, x_symbol, x_var, has_arg_src, has_arg_dst,
                    symbol_of_src, symbol_of_dst, var_occ_src, var_occ_dst,
                    zacc_h, zcnt_h)


def kernel(x_term, x_symbol, x_var, has_arg_src, has_arg_dst, symbol_of_src,
           symbol_of_dst, var_occ_src, var_occ_dst, Wl, bl, Wr, ln_g, ln_b,
           attn_in_w, attn_in_b, attn_out_w, attn_out_b, post_w, post_b):
    sums, cnts = _segment_sums(x_term, x_symbol, x_var, has_arg_src,
                               has_arg_dst, symbol_of_src, symbol_of_dst,
                               var_occ_src, var_occ_dst)
    Wr_sum = Wr[0] + Wr[1] + Wr[2] + Wr[3]
    b_sum = (bl[0] + bl[1] + bl[2] + bl[3]).reshape(1, HID)
    h, q, k, v = _dense_a(sums, cnts, x_term, Wl[:4], Wr_sum, b_sum,
                          ln_g.reshape(1, HID), ln_b.reshape(1, HID),
                          attn_in_w, attn_in_b.reshape(1, 3 * HID))
    return _attn(q, k, v, h, attn_out_w, attn_out_b.reshape(1, HID),
                 post_w, post_b.reshape(1, HID))


# speedup vs baseline: 1.0678x; 1.0631x over previous
import functools

import jax
import jax.numpy as jnp
from jax import lax
from jax.experimental import pallas as pl
from jax.experimental.pallas import tpu as pltpu
from jax.experimental.pallas import tpu_sc as plsc

HID = 256
HEADS = 4
DH = HID // HEADS
N_TERM = 4096
BLK = 256
N_BLKS = N_TERM // BLK
CNTW = 16


def _dense_a_body(sums_ref, cnts_ref, x_ref, wl_ref, wr_ref, bsum_ref,
                  lng_ref, lnb_ref, inw_ref, inb_ref,
                  h_ref, q_ref, k_ref, v_ref):
    x = x_ref[...]
    acc = x + bsum_ref[...]
    acc = acc + lax.dot_general(x, wr_ref[...], (((1,), (1,)), ((), ())),
                                preferred_element_type=jnp.float32)
    for r in range(4):
        c = jnp.transpose(cnts_ref[r, 0, :, 0:BLK])
        inv = 1.0 / jnp.maximum(c, 1.0)
        mean = sums_ref[r] * inv
        acc = acc + lax.dot_general(mean, wl_ref[r], (((1,), (1,)), ((), ())),
                                    preferred_element_type=jnp.float32)
    mu = jnp.mean(acc, axis=1, keepdims=True)
    var = jnp.mean(acc * acc, axis=1, keepdims=True) - mu * mu
    h = (acc - mu) * lax.rsqrt(var + 1e-5) * lng_ref[...] + lnb_ref[...]
    h_ref[...] = h
    qkv = lax.dot_general(h, inw_ref[...], (((1,), (1,)), ((), ())),
                          preferred_element_type=jnp.float32) + inb_ref[...]
    q_ref[...] = qkv[:, 0:HID]
    k_ref[...] = qkv[:, HID:2 * HID]
    v_ref[...] = qkv[:, 2 * HID:3 * HID]


def _dense_a(sums, cnts, x_term, Wl4, Wr_sum, b_sum, ln_g, ln_b, in_w, in_b):
    spec_row = pl.BlockSpec((BLK, HID), lambda i: (i, 0))
    return pl.pallas_call(
        _dense_a_body,
        grid=(N_BLKS,),
        in_specs=[
            pl.BlockSpec((4, BLK, HID), lambda i: (0, i, 0)),
            pl.BlockSpec((4, 1, 1, ACC_R), lambda i: (0, i, 0, 0)),
            spec_row,
            pl.BlockSpec((4, HID, HID), lambda i: (0, 0, 0)),
            pl.BlockSpec((HID, HID), lambda i: (0, 0)),
            pl.BlockSpec((1, HID), lambda i: (0, 0)),
            pl.BlockSpec((1, HID), lambda i: (0, 0)),
            pl.BlockSpec((1, HID), lambda i: (0, 0)),
            pl.BlockSpec((3 * HID, HID), lambda i: (0, 0)),
            pl.BlockSpec((1, 3 * HID), lambda i: (0, 0)),
        ],
        out_specs=[spec_row, spec_row, spec_row, spec_row],
        out_shape=[jax.ShapeDtypeStruct((N_TERM, HID), jnp.float32)] * 4,
    )(sums, cnts, x_term, Wl4, Wr_sum, b_sum, ln_g, ln_b, in_w, in_b)


def _attn_body(q_ref, k_ref, v_ref, h_ref, ow_ref, ob_ref, pw_ref, pb_ref,
               out_ref):
    q = q_ref[...]
    scale = 1.0 / (DH ** 0.5)
    outs = []
    for hh in range(HEADS):
        qh = q[:, hh * DH:(hh + 1) * DH] * scale
        kh = k_ref[:, hh * DH:(hh + 1) * DH]
        vh = v_ref[:, hh * DH:(hh + 1) * DH]
        s = lax.dot_general(qh, kh, (((1,), (1,)), ((), ())),
                            preferred_element_type=jnp.float32)
        m = jnp.max(s, axis=1, keepdims=True)
        p = jnp.exp(s - m)
        denom = jnp.sum(p, axis=1, keepdims=True)
        attn = p / denom
        outs.append(lax.dot_general(attn, vh, (((1,), (0,)), ((), ())),
                                    preferred_element_type=jnp.float32))
    o = jnp.concatenate(outs, axis=1)
    a = lax.dot_general(o, ow_ref[...], (((1,), (1,)), ((), ())),
                        preferred_element_type=jnp.float32) + ob_ref[...]
    f = lax.dot_general(a, pw_ref[...], (((1,), (1,)), ((), ())),
                        preferred_element_type=jnp.float32) + pb_ref[...]
    out_ref[...] = jnp.maximum(f, 0.0) + h_ref[...]


def _attn(q, k, v, h, out_w, out_b, post_w, post_b):
    spec_row = pl.BlockSpec((BLK, HID), lambda i: (i, 0))
    spec_full = pl.BlockSpec((N_TERM, HID), lambda i: (0, 0))
    return pl.pallas_call(
        _attn_body,
        grid=(N_BLKS,),
        in_specs=[
            spec_row, spec_full, spec_full, spec_row,
            pl.BlockSpec((HID, HID), lambda i: (0, 0)),
            pl.BlockSpec((1, HID), lambda i: (0, 0)),
            pl.BlockSpec((HID, HID), lambda i: (0, 0)),
            pl.BlockSpec((1, HID), lambda i: (0, 0)),
        ],
        out_specs=spec_row,
        out_shape=jax.ShapeDtypeStruct((N_TERM, HID), jnp.float32),
    )(q, k, v, h, out_w, out_b, post_w, post_b)


N_SUB = 16
TROWS = N_TERM // N_SUB
ACC_R = TROWS + 8
SCH = 4096
KSUB = 128


def _sc_body(x_term, x_symbol, x_var, has_src, has_dst, sym_src, sym_dst,
             var_src, var_dst, zacc_h, zcnt_h,
             sums_o, cnts_o,
             sbuf, dbuf, csrc, cdst, rows, acc, cacc, sem):
    sid = lax.axis_index("s")
    cid = lax.axis_index("c")
    lo = sid * TROWS
    lane = lax.iota(jnp.int32, 16)

    def process(r, table, srch, dsth, n_edges):
        pltpu.sync_copy(zacc_h, acc)
        pltpu.sync_copy(zcnt_h, cacc)
        def zbody(i, carry):
            csrc[pl.ds(i * 16, 16)] = jnp.zeros((16,), jnp.int32)
            return carry
        lax.fori_loop(0, (SCH + KSUB) // 16, zbody, 0)

        def schunk(s, carry):
            pltpu.sync_copy(srch.at[pl.ds(s * SCH, SCH)], sbuf)
            pltpu.sync_copy(dsth.at[pl.ds(s * SCH, SCH)], dbuf)

            def fbody(g, cur):
                dvec = dbuf[pl.ds(g * 16, 16)]
                svec = sbuf[pl.ds(g * 16, 16)]
                loc = dvec - lo
                m = (loc >= 0) & (loc < TROWS)
                locc = jnp.where(m, loc, TROWS)
                cnt, last = plsc.scan_count(locc)
                plsc.addupdate_scatter(cacc, [locc],
                                       cnt.astype(jnp.float32), mask=last)
                mi = m.astype(jnp.int32)
                pos = cur + plsc.cumsum(mi) - 1
                plsc.store_scatter(csrc, [pos], svec, mask=m)
                plsc.store_scatter(cdst, [pos], loc, mask=m)
                return cur + jnp.sum(mi)

            cur = lax.fori_loop(0, SCH // 16, fbody, 0)

            end = ((cur + KSUB - 1) // KSUB) * KSUB
            trash = jnp.full((16,), TROWS, jnp.int32)
            for p in range(KSUB // 16):
                pos = cur + p * 16 + lane
                plsc.store_scatter(cdst, [pos], trash, mask=pos < end)

            def kbody(k, carry):
                pltpu.async_copy(table.at[csrc.at[pl.ds(k * KSUB, KSUB)]],
                                 rows, sem).wait()
                base = k * KSUB

                def gbody(g, gcarry):
                    e0 = base + g * 16
                    dvec = cdst[pl.ds(e0, 16)]
                    for j in range(16):
                        d = dvec[j]
                        e = g * 16 + j
                        vals = [acc[d, pl.ds(q * 16, 16)]
                                + rows[e, pl.ds(q * 16, 16)]
                                for q in range(HID // 16)]
                        for q in range(HID // 16):
                            acc[d, pl.ds(q * 16, 16)] = vals[q]
                    return gcarry

                lax.fori_loop(0, KSUB // 16, gbody, 0)
                return carry

            lax.fori_loop(0, (cur + KSUB - 1) // KSUB, kbody, 0)
            return carry

        lax.fori_loop(0, n_edges // SCH, schunk, 0)

        pltpu.sync_copy(acc.at[pl.ds(0, TROWS)],
                        sums_o.at[r, pl.ds(lo, TROWS)])
        pltpu.sync_copy(cacc, cnts_o.at[r, sid, 0])

    @pl.when(cid == 0)
    def _():
        process(0, x_term, has_dst, has_src, has_src.shape[0])
        process(2, x_symbol, sym_dst, sym_src, sym_src.shape[0])

    @pl.when(cid == 1)
    def _():
        process(1, x_term, has_src, has_dst, has_src.shape[0])
        process(3, x_var, var_src, var_dst, var_src.shape[0])


_sc_call = functools.partial(
    pl.kernel,
    out_type=[
        jax.ShapeDtypeStruct((4, N_TERM, HID), jnp.float32),
        jax.ShapeDtypeStruct((4, N_SUB, 1, ACC_R), jnp.float32),
    ],
    mesh=plsc.VectorSubcoreMesh(core_axis_name="c", subcore_axis_name="s"),
    compiler_params=pltpu.CompilerParams(needs_layout_passes=False),
    scratch_types=[
        pltpu.VMEM((SCH,), jnp.int32),
        pltpu.VMEM((SCH,), jnp.int32),
        pltpu.VMEM((SCH + KSUB,), jnp.int32),
        pltpu.VMEM((SCH + KSUB,), jnp.int32),
        pltpu.VMEM((KSUB, HID), jnp.float32),
        pltpu.VMEM((ACC_R, HID), jnp.float32),
        pltpu.VMEM((ACC_R,), jnp.float32),
        pltpu.SemaphoreType.DMA,
    ],
)(_sc_body)


def _segment_sums(x_term, x_symbol, x_var, has_arg_src, has_arg_dst,
                  symbol_of_src, symbol_of_dst, var_occ_src, var_occ_dst):
    zacc_h = jnp.zeros((ACC_R, HID), jnp.float32)
    zcnt_h = jnp.zeros((ACC_R,), jnp.float32)
    return _sc_call(x_term, x_symbol, x_var, has_arg_src, has_arg_dst,
                    symbol_of_src, symbol_of_dst, var_occ_src, var_occ_dst,
                    zacc_h, zcnt_h)


def kernel(x_term, x_symbol, x_var, has_arg_src, has_arg_dst, symbol_of_src,
           symbol_of_dst, var_occ_src, var_occ_dst, Wl, bl, Wr, ln_g, ln_b,
           attn_in_w, attn_in_b, attn_out_w, attn_out_b, post_w, post_b):
    sums, cnts = _segment_sums(x_term, x_symbol, x_var, has_arg_src,
                               has_arg_dst, symbol_of_src, symbol_of_dst,
                               var_occ_src, var_occ_dst)
    Wr_sum = Wr[0] + Wr[1] + Wr[2] + Wr[3]
    b_sum = (bl[0] + bl[1] + bl[2] + bl[3]).reshape(1, HID)
    h, q, k, v = _dense_a(sums, cnts, x_term, Wl[:4], Wr_sum, b_sum,
                          ln_g.reshape(1, HID), ln_b.reshape(1, HID),
                          attn_in_w, attn_in_b.reshape(1, 3 * HID))
    return _attn(q, k, v, h, attn_out_w, attn_out_b.reshape(1, HID),
                 post_w, post_b.reshape(1, HID))
